# baseline (speedup 1.0000x reference)
import functools

import jax
import jax.numpy as jnp
from jax import lax
from jax.experimental import pallas as pl
from jax.experimental.pallas import tpu as pltpu
from jax.experimental.pallas import tpu_sc as plsc

E = 8
D = 2048
F = 8192
T = 2048
CAP = 256
LANES = 128
RB = 256
FB = 1024
NC, NS, L = 2, 16, 16
NW = NC * NS
SPW = T // NW
CHUNK = 32


def _router_body(x_ref, wr_ref, br_ref, slot_ref, gate_ref, carry_ref):
    i = pl.program_id(0)

    @pl.when(i == 0)
    def _():
        carry_ref[...] = jnp.zeros_like(carry_ref)

    xb = x_ref[...]
    logits = jnp.dot(xb, wr_ref[...],
                     preferred_element_type=jnp.float32) + br_ref[...]
    m = jnp.max(logits, axis=1, keepdims=True)
    ex = jnp.exp(logits - m)
    p = ex / jnp.sum(ex, axis=1, keepdims=True)
    pm = jnp.max(p, axis=1, keepdims=True)
    lane = lax.broadcasted_iota(jnp.int32, (RB, LANES), 1)
    idx = jnp.min(jnp.where(p >= pm, lane, LANES), axis=1, keepdims=True)
    oh = (lane == idx).astype(jnp.float32)
    row = lax.broadcasted_iota(jnp.int32, (RB, RB), 0)
    col = lax.broadcasted_iota(jnp.int32, (RB, RB), 1)
    tri = (row >= col).astype(jnp.float32)
    pos = jnp.dot(tri, oh, preferred_element_type=jnp.float32)
    pos = pos + carry_ref[0:1, :]
    carry_ref[0:1, :] = carry_ref[0:1, :] + jnp.sum(oh, axis=0, keepdims=True)
    pos_t = jnp.sum(pos * oh, axis=1, keepdims=True).astype(jnp.int32)
    kept = pos_t < CAP
    slot = jnp.where(kept, idx * CAP + pos_t, 0)
    gate = jnp.where(kept, pm, 0.0)
    slot_ref[...] = jnp.broadcast_to(slot, (RB, LANES))
    gate_ref[...] = jnp.broadcast_to(gate, (RB, LANES))


def _router(xf, wr_pad, br_pad):
    return pl.pallas_call(
        _router_body,
        grid=(T // RB,),
        in_specs=[
            pl.BlockSpec((RB, D), lambda i: (i, 0)),
            pl.BlockSpec((D, LANES), lambda i: (0, 0)),
            pl.BlockSpec((1, LANES), lambda i: (0, 0)),
        ],
        out_specs=[
            pl.BlockSpec((RB, LANES), lambda i: (i, 0)),
            pl.BlockSpec((RB, LANES), lambda i: (i, 0)),
        ],
        out_shape=[
            jax.ShapeDtypeStruct((T, LANES), jnp.int32),
            jax.ShapeDtypeStruct((T, LANES), jnp.float32),
        ],
        scratch_shapes=[pltpu.VMEM((8, LANES), jnp.float32)],
    )(xf, wr_pad, br_pad)


def _dispatch_body(sft_hbm, gate_hbm, x_hbm, ex_hbm, gfs_hbm,
                   sft_v, gate_v, tfs_v, idx_v, gfs_v, rows_v, sem):
    wid = lax.axis_index("s") * NC + lax.axis_index("c")
    pltpu.sync_copy(sft_hbm, sft_v)
    pltpu.sync_copy(gate_hbm, gate_v)

    def init(i, c):
        tfs_v[pl.ds(i * L, L)] = jnp.zeros((L,), jnp.int32)
        return c

    lax.fori_loop(0, T // L, init, 0)

    def scat(i, c):
        toks = lax.iota(jnp.int32, L) + i * L
        slots = sft_v[pl.ds(i * L, L)]
        plsc.store_scatter(tfs_v, [slots], toks)
        return c

    lax.fori_loop(0, T // L, scat, 0)

    base = wid * SPW

    def gf(k, c):
        toks = tfs_v[pl.ds(base + k * L, L)]
        gfs_v[pl.ds(k * L, L)] = plsc.load_gather(gate_v, [toks])
        return c

    lax.fori_loop(0, SPW // L, gf, 0)
    pltpu.sync_copy(gfs_v, gfs_hbm.at[pl.ds(base, SPW)])

    for c in range(SPW // CHUNK):
        for k in range(CHUNK // L):
            idx_v[pl.ds(k * L, L)] = tfs_v[pl.ds(base + c * CHUNK + k * L, L)]
        pltpu.async_copy(x_hbm.at[idx_v], rows_v, sem).wait()
        pltpu.sync_copy(rows_v, ex_hbm.at[pl.ds(base + c * CHUNK, CHUNK)])


def _gelu(h):
    return 0.5 * h * (1.0 + lax.erf(h * 0.7071067811865476))


def _ffn_body(x_ref, g_ref, w1_ref, b1_ref, w2_ref, b2_ref, out_ref, acc_ref):
    f = pl.program_id(1)

    @pl.when(f == 0)
    def _():
        acc_ref[...] = jnp.zeros_like(acc_ref)

    gcol = g_ref[0, :, 0:1]
    xg = x_ref[0] * gcol
    h = jnp.dot(xg.astype(jnp.bfloat16), w1_ref[0].astype(jnp.bfloat16),
                preferred_element_type=jnp.float32) + b1_ref[0]
    h = _gelu(h)
    acc_ref[...] += jnp.dot(h.astype(jnp.bfloat16),
                            w2_ref[0].astype(jnp.bfloat16),
                            preferred_element_type=jnp.float32)

    @pl.when(f == F // FB - 1)
    def _():
        out_ref[0] = (acc_ref[...] + b2_ref[0]) * gcol


def _ffn(ex, g3, W1, b1, W2, b2):
    return pl.pallas_call(
        _ffn_body,
        grid=(E, F // FB),
        in_specs=[
            pl.BlockSpec((1, CAP, D), lambda e, f: (e, 0, 0)),
            pl.BlockSpec((1, CAP, LANES), lambda e, f: (e, 0, 0)),
            pl.BlockSpec((1, D, FB), lambda e, f: (e, 0, f)),
            pl.BlockSpec((1, 1, FB), lambda e, f: (e, 0, f)),
            pl.BlockSpec((1, FB, D), lambda e, f: (e, f, 0)),
            pl.BlockSpec((1, 1, D), lambda e, f: (e, 0, 0)),
        ],
        out_specs=pl.BlockSpec((1, CAP, D), lambda e, f: (e, 0, 0)),
        out_shape=jax.ShapeDtypeStruct((E, CAP, D), jnp.float32),
        scratch_shapes=[pltpu.VMEM((CAP, D), jnp.float32)],
    )(ex, g3, W1, b1, W2, b2)


def _combine_body(sft_hbm, y_hbm, out_hbm, idx_v, rows_v, sem):
    wid = lax.axis_index("s") * NC + lax.axis_index("c")
    base = wid * SPW
    for c in range(SPW // CHUNK):
        pltpu.sync_copy(sft_hbm.at[pl.ds(base + c * CHUNK, CHUNK)], idx_v)
        pltpu.async_copy(y_hbm.at[idx_v], rows_v, sem).wait()
        pltpu.sync_copy(rows_v, out_hbm.at[pl.ds(base + c * CHUNK, CHUNK)])


@functools.lru_cache(maxsize=None)
def _sc_kernels():
    mesh = plsc.VectorSubcoreMesh(core_axis_name="c", subcore_axis_name="s",
                                  num_cores=NC, num_subcores=NS)
    params = pltpu.CompilerParams(needs_layout_passes=False)
    dispatch = pl.kernel(
        _dispatch_body,
        compiler_params=params,
        out_type=[
            jax.ShapeDtypeStruct((T, D), jnp.float32),
            jax.ShapeDtypeStruct((T,), jnp.float32),
        ],
        mesh=mesh,
        scratch_types=[
            pltpu.VMEM((T,), jnp.int32),
            pltpu.VMEM((T,), jnp.float32),
            pltpu.VMEM((T,), jnp.int32),
            pltpu.VMEM((CHUNK,), jnp.int32),
            pltpu.VMEM((SPW,), jnp.float32),
            pltpu.VMEM((CHUNK, D), jnp.float32),
            pltpu.SemaphoreType.DMA,
        ],
    )
    combine = pl.kernel(
        _combine_body,
        compiler_params=params,
        out_type=jax.ShapeDtypeStruct((T, D), jnp.float32),
        mesh=mesh,
        scratch_types=[
            pltpu.VMEM((CHUNK,), jnp.int32),
            pltpu.VMEM((CHUNK, D), jnp.float32),
            pltpu.SemaphoreType.DMA,
        ],
    )
    return dispatch, combine


def kernel(x, W_router, b_router, W1, b1, W2, b2):
    _dispatch, _combine = _sc_kernels()
    xf = x.reshape(T, D)
    wr_pad = jnp.zeros((D, LANES), jnp.float32).at[:, :E].set(W_router)
    br_pad = jnp.full((LANES,), -1e30, jnp.float32).at[:E].set(b_router)
    br_pad = br_pad.reshape(1, LANES)
    slot128, gate128 = _router(xf, wr_pad, br_pad)
    slot = slot128[:, 0]
    gate = gate128[:, 0]
    ex, gfs = _dispatch(slot, gate, xf)
    g3 = jnp.broadcast_to(gfs.reshape(E, CAP, 1), (E, CAP, LANES))
    y = _ffn(ex.reshape(E, CAP, D), g3, W1,
             b1.reshape(E, 1, F), W2, b2.reshape(E, 1, D))
    out = _combine(slot, y.reshape(T, D))
    return out.reshape(x.shape)


# speedup vs baseline: 1.0025x; 1.0025x over previous
import functools

import jax
import jax.numpy as jnp
from jax import lax
from jax.experimental import pallas as pl
from jax.experimental.pallas import tpu as pltpu
from jax.experimental.pallas import tpu_sc as plsc

E = 8
D = 2048
F = 8192
T = 2048
CAP = 256
LANES = 128
RB = 256
FB = 1024
NC, NS, L = 2, 16, 16
NW = NC * NS
SPW = T // NW
CHUNK = 16
NCHUNK = SPW // CHUNK


def _router_body(x_ref, wr_ref, br_ref, slot_ref, gate_ref, carry_ref):
    i = pl.program_id(0)

    @pl.when(i == 0)
    def _():
        carry_ref[...] = jnp.zeros_like(carry_ref)

    xb = x_ref[...]
    logits = jnp.dot(xb, wr_ref[...],
                     preferred_element_type=jnp.float32) + br_ref[...]
    m = jnp.max(logits, axis=1, keepdims=True)
    ex = jnp.exp(logits - m)
    p = ex / jnp.sum(ex, axis=1, keepdims=True)
    pm = jnp.max(p, axis=1, keepdims=True)
    lane = lax.broadcasted_iota(jnp.int32, (RB, LANES), 1)
    idx = jnp.min(jnp.where(p >= pm, lane, LANES), axis=1, keepdims=True)
    oh = (lane == idx).astype(jnp.float32)
    row = lax.broadcasted_iota(jnp.int32, (RB, RB), 0)
    col = lax.broadcasted_iota(jnp.int32, (RB, RB), 1)
    tri = (row >= col).astype(jnp.float32)
    pos = jnp.dot(tri, oh, preferred_element_type=jnp.float32)
    pos = pos + carry_ref[0:1, :]
    carry_ref[0:1, :] = carry_ref[0:1, :] + jnp.sum(oh, axis=0, keepdims=True)
    pos_t = jnp.sum(pos * oh, axis=1, keepdims=True).astype(jnp.int32)
    kept = pos_t < CAP
    slot = jnp.where(kept, idx * CAP + pos_t, 0)
    gate = jnp.where(kept, pm, 0.0)
    slot_ref[...] = jnp.broadcast_to(slot, (RB, LANES))
    gate_ref[...] = jnp.broadcast_to(gate, (RB, LANES))


def _router(xf, wr_pad, br_pad):
    return pl.pallas_call(
        _router_body,
        grid=(T // RB,),
        in_specs=[
            pl.BlockSpec((RB, D), lambda i: (i, 0)),
            pl.BlockSpec((D, LANES), lambda i: (0, 0)),
            pl.BlockSpec((1, LANES), lambda i: (0, 0)),
        ],
        out_specs=[
            pl.BlockSpec((RB, LANES), lambda i: (i, 0)),
            pl.BlockSpec((RB, LANES), lambda i: (i, 0)),
        ],
        out_shape=[
            jax.ShapeDtypeStruct((T, LANES), jnp.int32),
            jax.ShapeDtypeStruct((T, LANES), jnp.float32),
        ],
        scratch_shapes=[pltpu.VMEM((8, LANES), jnp.float32)],
    )(xf, wr_pad, br_pad)


def _dispatch_body(sft_hbm, gate_hbm, x_hbm, ex_hbm, gfs_hbm,
                   sft_v, gate_v, tfs_v, idx_v, gfs_v, rows_v, sem_a, sem_b):
    wid = lax.axis_index("s") * NC + lax.axis_index("c")
    pltpu.sync_copy(sft_hbm, sft_v)
    pltpu.sync_copy(gate_hbm, gate_v)

    UNROLL = 8

    def init(i, c):
        for u in range(UNROLL):
            tfs_v[pl.ds((i * UNROLL + u) * L, L)] = jnp.zeros((L,), jnp.int32)
        return c

    lax.fori_loop(0, T // L // UNROLL, init, 0)

    def scat(i, c):
        for u in range(UNROLL):
            j = i * UNROLL + u
            toks = lax.iota(jnp.int32, L) + j * L
            slots = sft_v[pl.ds(j * L, L)]
            plsc.store_scatter(tfs_v, [slots], toks)
        return c

    lax.fori_loop(0, T // L // UNROLL, scat, 0)

    base = wid * SPW

    for k in range(SPW // L):
        toks = tfs_v[pl.ds(base + k * L, L)]
        gfs_v[pl.ds(k * L, L)] = plsc.load_gather(gate_v, [toks])
        idx_v[k, :] = toks
    pltpu.sync_copy(gfs_v, gfs_hbm.at[pl.ds(base, SPW)])

    sems = [sem_a, sem_b]
    copies = [
        pltpu.make_async_copy(x_hbm.at[idx_v.at[c]], rows_v.at[c % 2],
                              sems[c % 2])
        for c in range(NCHUNK)
    ]
    copies[0].start()
    for c in range(NCHUNK):
        copies[c].wait()
        if c + 1 < NCHUNK:
            copies[c + 1].start()
        pltpu.sync_copy(rows_v.at[c % 2],
                        ex_hbm.at[pl.ds(base + c * CHUNK, CHUNK)])


def _gelu(h):
    return 0.5 * h * (1.0 + lax.erf(h * 0.7071067811865476))


def _ffn_body(x_ref, g_ref, w1_ref, b1_ref, w2_ref, b2_ref, out_ref, acc_ref):
    f = pl.program_id(1)

    @pl.when(f == 0)
    def _():
        acc_ref[...] = jnp.zeros_like(acc_ref)

    gcol = g_ref[0, :, 0:1]
    xg = x_ref[0] * gcol
    h = jnp.dot(xg, w1_ref[0], preferred_element_type=jnp.float32) + b1_ref[0]
    h = _gelu(h)
    acc_ref[...] += jnp.dot(h, w2_ref[0], preferred_element_type=jnp.float32)

    @pl.when(f == F // FB - 1)
    def _():
        out_ref[0] = (acc_ref[...] + b2_ref[0]) * gcol


def _ffn(ex, g3, W1, b1, W2, b2):
    return pl.pallas_call(
        _ffn_body,
        grid=(E, F // FB),
        in_specs=[
            pl.BlockSpec((1, CAP, D), lambda e, f: (e, 0, 0)),
            pl.BlockSpec((1, CAP, LANES), lambda e, f: (e, 0, 0)),
            pl.BlockSpec((1, D, FB), lambda e, f: (e, 0, f)),
            pl.BlockSpec((1, 1, FB), lambda e, f: (e, 0, f)),
            pl.BlockSpec((1, FB, D), lambda e, f: (e, f, 0)),
            pl.BlockSpec((1, 1, D), lambda e, f: (e, 0, 0)),
        ],
        out_specs=pl.BlockSpec((1, CAP, D), lambda e, f: (e, 0, 0)),
        out_shape=jax.ShapeDtypeStruct((E, CAP, D), jnp.float32),
        scratch_shapes=[pltpu.VMEM((CAP, D), jnp.float32)],
    )(ex, g3, W1, b1, W2, b2)


def _combine_body(sft_hbm, y_hbm, out_hbm, idx_v, rows_v, sem_a, sem_b):
    wid = lax.axis_index("s") * NC + lax.axis_index("c")
    base = wid * SPW
    for c in range(NCHUNK):
        pltpu.sync_copy(sft_hbm.at[pl.ds(base + c * CHUNK, CHUNK)],
                        idx_v.at[c])
    sems = [sem_a, sem_b]
    copies = [
        pltpu.make_async_copy(y_hbm.at[idx_v.at[c]], rows_v.at[c % 2],
                              sems[c % 2])
        for c in range(NCHUNK)
    ]
    copies[0].start()
    for c in range(NCHUNK):
        copies[c].wait()
        if c + 1 < NCHUNK:
            copies[c + 1].start()
        pltpu.sync_copy(rows_v.at[c % 2],
                        out_hbm.at[pl.ds(base + c * CHUNK, CHUNK)])


@functools.lru_cache(maxsize=None)
def _sc_kernels():
    mesh = plsc.VectorSubcoreMesh(core_axis_name="c", subcore_axis_name="s",
                                  num_cores=NC, num_subcores=NS)
    params = pltpu.CompilerParams(needs_layout_passes=False)
    dispatch = pl.kernel(
        _dispatch_body,
        compiler_params=params,
        out_type=[
            jax.ShapeDtypeStruct((T, D), jnp.float32),
            jax.ShapeDtypeStruct((T,), jnp.float32),
        ],
        mesh=mesh,
        scratch_types=[
            pltpu.VMEM((T,), jnp.int32),
            pltpu.VMEM((T,), jnp.float32),
            pltpu.VMEM((T,), jnp.int32),
            pltpu.VMEM((NCHUNK, L), jnp.int32),
            pltpu.VMEM((SPW,), jnp.float32),
            pltpu.VMEM((2, CHUNK, D), jnp.float32),
            pltpu.SemaphoreType.DMA,
            pltpu.SemaphoreType.DMA,
        ],
    )
    combine = pl.kernel(
        _combine_body,
        compiler_params=params,
        out_type=jax.ShapeDtypeStruct((T, D), jnp.float32),
        mesh=mesh,
        scratch_types=[
            pltpu.VMEM((NCHUNK, L), jnp.int32),
            pltpu.VMEM((2, CHUNK, D), jnp.float32),
            pltpu.SemaphoreType.DMA,
            pltpu.SemaphoreType.DMA,
        ],
    )
    return dispatch, combine


def kernel(x, W_router, b_router, W1, b1, W2, b2):
    _dispatch, _combine = _sc_kernels()
    xf = x.reshape(T, D)
    wr_pad = jnp.zeros((D, LANES), jnp.float32).at[:, :E].set(W_router)
    br_pad = jnp.full((LANES,), -1e30, jnp.float32).at[:E].set(b_router)
    br_pad = br_pad.reshape(1, LANES)
    slot128, gate128 = _router(xf, wr_pad, br_pad)
    slot = slot128[:, 0]
    gate = gate128[:, 0]
    ex, gfs = _dispatch(slot, gate, xf)
    g3 = jnp.broadcast_to(gfs.reshape(E, CAP, 1), (E, CAP, LANES))
    y = _ffn(ex.reshape(E, CAP, D), g3, W1,
             b1.reshape(E, 1, F), W2, b2.reshape(E, 1, D))
    out = _combine(slot, y.reshape(T, D))
    return out.reshape(x.shape)
